# baseline (speedup 1.0000x reference)
import jax
import jax.numpy as jnp
from jax.experimental import pallas as pl
from jax.experimental.pallas import tpu as pltpu

_D = 64
_K = 1024
_B = 32
_R = 1024
_G = 72


def _vq_body(xt_ref, et_ref, qt_ref, idx_ref, perp_ref, g_ref, counts_ref):
    i = pl.program_id(0)
    xt = xt_ref[0]
    et = et_ref[...]

    @pl.when(i == 0)
    def _mkg():
        r = jax.lax.broadcasted_iota(jnp.int32, (_G - _D, _K), 0)
        c = jax.lax.broadcasted_iota(jnp.int32, (_G - _D, _K), 1)
        w = jnp.where(r == 0, c >> 5,
                      jnp.where(r == 1, c & 31,
                                jnp.where(r == 2, 1, 0)))
        g_ref[0:_D, :] = et
        g_ref[_D:_G, :] = w.astype(jnp.float32)

    a_sq = jnp.sum(xt * xt, axis=0, keepdims=True)
    ab = 2.0 * jax.lax.dot_general(
        et, xt, (((0,), (0,)), ((), ())),
        preferred_element_type=jnp.float32)
    ones8 = jnp.ones((_D, 8), jnp.float32)
    b_sq = jax.lax.dot_general(
        et * et, ones8, (((0,), (0,)), ((), ())),
        precision=jax.lax.Precision.HIGHEST,
        preferred_element_type=jnp.float32)[:, 0:1]
    dist = (a_sq - ab) + b_sq

    dmin = jnp.min(dist, axis=0, keepdims=True)
    eqf = (dist == dmin).astype(jnp.float32)
    mm = jax.lax.dot_general(
        g_ref[...], eqf, (((1,), (0,)), ((), ())),
        preferred_element_type=jnp.float32)
    qt = mm[0:_D, :]
    hi = mm[_D:_D + 1, :]
    lo = mm[_D + 1:_D + 2, :]
    cnt = mm[_D + 2:_D + 3, :]
    tie = jnp.max(cnt) > 1.5

    def _store_idx(row):
        t = jnp.broadcast_to(row.reshape(1, _K), (_B, _K))
        rows = jax.lax.broadcasted_iota(jnp.int32, (_B, _K), 0)
        prev = jnp.where(i == 0, jnp.zeros((_B, _K), jnp.int32), idx_ref[...])
        idx_ref[...] = jnp.where(rows == i, t, prev)

    _store_idx((hi * 32.0 + lo)[0, :].astype(jnp.int32))
    qt_ref[0] = xt + (qt - xt)

    blk_counts = jax.lax.dot_general(
        eqf, jnp.ones((_R, 8), jnp.float32), (((1,), (0,)), ((), ())),
        preferred_element_type=jnp.float32)[:, 0:1]

    @pl.when(i == 0)
    def _init():
        counts_ref[...] = blk_counts

    @pl.when(i > 0)
    def _acc():
        counts_ref[...] += blk_counts

    @pl.when(tie)
    def _slow():
        ids = jax.lax.broadcasted_iota(jnp.int32, (_K, _R), 0)
        idxv = jnp.min(jnp.where(dist == dmin, ids, _K), axis=0,
                       keepdims=True)
        _store_idx(idxv[0, :])
        oh = (ids == idxv).astype(jnp.float32)
        q2 = jax.lax.dot_general(
            et, oh, (((1,), (0,)), ((), ())),
            preferred_element_type=jnp.float32)
        qt_ref[0] = xt + (q2 - xt)
        oh_counts = jax.lax.dot_general(
            oh, jnp.ones((_R, 8), jnp.float32), (((1,), (0,)), ((), ())),
            preferred_element_type=jnp.float32)[:, 0:1]
        counts_ref[...] += oh_counts - blk_counts

    @pl.when(i == _B - 1)
    def _final():
        p = counts_ref[...] * (1.0 / (_B * _R))
        ent = -jnp.sum(p * jnp.log(p + 1e-10))
        perp_ref[0, 0] = jnp.exp(ent)


def kernel(inputs, embeddings):
    xt = jnp.transpose(inputs, (0, 2, 1))
    et = embeddings.T
    qt, idx, perp = pl.pallas_call(
        _vq_body,
        grid=(_B,),
        in_specs=[
            pl.BlockSpec((1, _D, _R), lambda i: (i, 0, 0)),
            pl.BlockSpec((_D, _K), lambda i: (0, 0)),
        ],
        out_specs=[
            pl.BlockSpec((1, _D, _R), lambda i: (i, 0, 0)),
            pl.BlockSpec((_B, _K), lambda i: (0, 0)),
            pl.BlockSpec(memory_space=pltpu.SMEM),
        ],
        out_shape=[
            jax.ShapeDtypeStruct((_B, _D, _R), jnp.float32),
            jax.ShapeDtypeStruct((_B, _K), jnp.int32),
            jax.ShapeDtypeStruct((1, 1), jnp.float32),
        ],
        scratch_shapes=[
            pltpu.VMEM((_G, _K), jnp.float32),
            pltpu.VMEM((_K, 1), jnp.float32),
        ],
    )(xt, et)
    quantized_st = jnp.transpose(qt, (0, 2, 1))
    return (quantized_st, idx, perp[0, 0])


# speedup vs baseline: 1.3672x; 1.3672x over previous
import jax
import jax.numpy as jnp
from jax.experimental import pallas as pl
from jax.experimental.pallas import tpu as pltpu

_D = 64
_K = 1024
_B = 32
_R = 1024
_G = 72


def _vq_body(xt_ref, et_ref, qt_ref, idx_ref, perp_ref, g_ref, counts_ref):
    i = pl.program_id(0)
    xt = xt_ref[0]
    et = et_ref[...]

    @pl.when(i == 0)
    def _mkg():
        r = jax.lax.broadcasted_iota(jnp.int32, (_G - _D, _K), 0)
        c = jax.lax.broadcasted_iota(jnp.int32, (_G - _D, _K), 1)
        w = jnp.where(r == 0, c >> 5,
                      jnp.where(r == 1, c & 31,
                                jnp.where(r == 2, 1, 0)))
        g_ref[0:_D, :] = et
        g_ref[_D:_G, :] = w.astype(jnp.float32)

    a_sq = jnp.sum(xt * xt, axis=0, keepdims=True)
    ab = 2.0 * jax.lax.dot_general(
        et, xt, (((0,), (0,)), ((), ())),
        preferred_element_type=jnp.float32)
    ones8 = jnp.ones((_D, 8), jnp.float32)
    b_sq = jax.lax.dot_general(
        et * et, ones8, (((0,), (0,)), ((), ())),
        precision=jax.lax.Precision.HIGHEST,
        preferred_element_type=jnp.float32)[:, 0:1]
    dist = (a_sq - ab) + b_sq

    dmin = jnp.min(dist, axis=0, keepdims=True)
    eqf = (dist == dmin).astype(jnp.float32)
    mm = jax.lax.dot_general(
        g_ref[...], eqf, (((1,), (0,)), ((), ())),
        preferred_element_type=jnp.float32)
    qt = mm[0:_D, :]
    hi = mm[_D:_D + 1, :]
    lo = mm[_D + 1:_D + 2, :]
    cnt = mm[_D + 2:_D + 3, :]
    tie = jnp.max(cnt) > 1.5

    def _store_idx(row):
        t = jnp.broadcast_to(row.reshape(1, _K), (_B, _K))
        rows = jax.lax.broadcasted_iota(jnp.int32, (_B, _K), 0)
        prev = jnp.where(i == 0, jnp.zeros((_B, _K), jnp.int32), idx_ref[...])
        idx_ref[...] = jnp.where(rows == i, t, prev)

    _store_idx((hi * 32.0 + lo)[0, :].astype(jnp.int32))
    qt_ref[0] = xt + (qt - xt)

    blk_counts = jnp.sum(eqf, axis=1, keepdims=True)

    @pl.when(i == 0)
    def _init():
        counts_ref[...] = blk_counts

    @pl.when(i > 0)
    def _acc():
        counts_ref[...] += blk_counts

    @pl.when(tie)
    def _slow():
        ids = jax.lax.broadcasted_iota(jnp.int32, (_K, _R), 0)
        idxv = jnp.min(jnp.where(dist == dmin, ids, _K), axis=0,
                       keepdims=True)
        _store_idx(idxv[0, :])
        oh = (ids == idxv).astype(jnp.float32)
        q2 = jax.lax.dot_general(
            et, oh, (((1,), (0,)), ((), ())),
            preferred_element_type=jnp.float32)
        qt_ref[0] = xt + (q2 - xt)
        counts_ref[...] += jnp.sum(oh, axis=1, keepdims=True) - blk_counts

    @pl.when(i == _B - 1)
    def _final():
        p = counts_ref[...] * (1.0 / (_B * _R))
        ent = -jnp.sum(p * jnp.log(p + 1e-10))
        perp_ref[0, 0] = jnp.exp(ent)


def kernel(inputs, embeddings):
    xt = jnp.transpose(inputs, (0, 2, 1))
    et = embeddings.T
    qt, idx, perp = pl.pallas_call(
        _vq_body,
        grid=(_B,),
        in_specs=[
            pl.BlockSpec((1, _D, _R), lambda i: (i, 0, 0)),
            pl.BlockSpec((_D, _K), lambda i: (0, 0)),
        ],
        out_specs=[
            pl.BlockSpec((1, _D, _R), lambda i: (i, 0, 0)),
            pl.BlockSpec((_B, _K), lambda i: (0, 0)),
            pl.BlockSpec(memory_space=pltpu.SMEM),
        ],
        out_shape=[
            jax.ShapeDtypeStruct((_B, _D, _R), jnp.float32),
            jax.ShapeDtypeStruct((_B, _K), jnp.int32),
            jax.ShapeDtypeStruct((1, 1), jnp.float32),
        ],
        scratch_shapes=[
            pltpu.VMEM((_G, _K), jnp.float32),
            pltpu.VMEM((_K, 1), jnp.float32),
        ],
    )(xt, et)
    quantized_st = jnp.transpose(qt, (0, 2, 1))
    return (quantized_st, idx, perp[0, 0])
